# Initial kernel scaffold; baseline (speedup 1.0000x reference)
import jax
import jax.numpy as jnp
from jax.experimental import pallas as pl


def kernel(x, W_qkv, b_qkv, W_proj, b_proj):
    raise NotImplementedError("write your pallas kernel here")


# speedup vs baseline: 1.9454x; 1.9454x over previous
import functools

import jax
import jax.numpy as jnp
from jax.experimental import pallas as pl
from jax.experimental.pallas import tpu as pltpu

DIM = 512
NUM_HEADS = 8
HEAD_DIM = DIM // NUM_HEADS
N_WIN = 8
TOPK = 4
TAU = 2.0
VTH = 1.0
WIN = 128


def _qkv_kernel(x_ref, w_ref, b_ref, q_ref, k_ref, v_ref):
    xb = x_ref[0].astype(jnp.bfloat16)
    outs = (q_ref, k_ref, v_ref)
    for n in range(3):
        z = jax.lax.dot_general(
            xb, w_ref[:, n * DIM:(n + 1) * DIM],
            (((1,), (0,)), ((), ())),
            preferred_element_type=jnp.float32,
        )
        z = z + b_ref[0, n * DIM:(n + 1) * DIM][None, :]
        spk = (z * (1.0 / TAU) >= VTH)
        outs[n][0] = spk.astype(jnp.bfloat16)


def _attn_kernel(idx_ref, q_ref, k_ref, v_ref, wp_ref, bp_ref, o_ref,
                 kg_s, vg_s, at_s):
    b = pl.program_id(0) % 2
    scale = HEAD_DIM ** -0.5
    for w in range(N_WIN):
        for j in range(TOPK):
            src = idx_ref[b, w, j]
            kg_s[j * WIN:(j + 1) * WIN, :] = k_ref[0, pl.ds(src * WIN, WIN), :]
            vg_s[j * WIN:(j + 1) * WIN, :] = v_ref[0, pl.ds(src * WIN, WIN), :]
        for h in range(NUM_HEADS):
            c0 = h * HEAD_DIM
            qh = q_ref[0, w * WIN:(w + 1) * WIN, c0:c0 + HEAD_DIM]
            kh = kg_s[:, c0:c0 + HEAD_DIM]
            s = jax.lax.dot_general(
                qh, kh, (((1,), (1,)), ((), ())),
                preferred_element_type=jnp.float32,
            ) * scale
            m = jnp.max(s, axis=1, keepdims=True)
            p = jnp.exp(s - m)
            p = p / jnp.sum(p, axis=1, keepdims=True)
            at_s[:, c0:c0 + HEAD_DIM] = jax.lax.dot_general(
                p.astype(jnp.bfloat16), vg_s[:, c0:c0 + HEAD_DIM],
                (((1,), (0,)), ((), ())),
                preferred_element_type=jnp.float32,
            )
        z = jax.lax.dot_general(
            at_s[...].astype(jnp.bfloat16), wp_ref[...],
            (((1,), (0,)), ((), ())),
            preferred_element_type=jnp.float32,
        )
        z = z + bp_ref[0][None, :]
        spk = (z * (1.0 / TAU) >= VTH)
        o_ref[0, w * WIN:(w + 1) * WIN, :] = spk.astype(jnp.float32)


def _routing_indices(x):
    T, B, L, C = x.shape
    x_sum = x.sum(axis=0)
    region_feat = x_sum.reshape(B, N_WIN, L // N_WIN, C).sum(axis=2)
    attn_r = region_feat @ jnp.swapaxes(region_feat, -2, -1)
    _, idx = jax.lax.top_k(attn_r, TOPK)
    return idx.astype(jnp.int32)


def kernel(x, W_qkv, b_qkv, W_proj, b_proj):
    T, B, L, C = x.shape
    TB = T * B
    idx = _routing_indices(x)

    xr = x.reshape(TB, L, C)
    wqkvT = W_qkv.T.astype(jnp.bfloat16)
    bqkv = b_qkv.reshape(1, 3 * C)
    wpT = W_proj.T.astype(jnp.bfloat16)
    bp = b_proj.reshape(1, C)

    qkv_shape = jax.ShapeDtypeStruct((TB, L, C), jnp.bfloat16)
    q, k, v = pl.pallas_call(
        _qkv_kernel,
        grid=(TB,),
        in_specs=[
            pl.BlockSpec((1, L, C), lambda i: (i, 0, 0)),
            pl.BlockSpec((C, 3 * C), lambda i: (0, 0)),
            pl.BlockSpec((1, 3 * C), lambda i: (0, 0)),
        ],
        out_specs=[
            pl.BlockSpec((1, L, C), lambda i: (i, 0, 0)),
            pl.BlockSpec((1, L, C), lambda i: (i, 0, 0)),
            pl.BlockSpec((1, L, C), lambda i: (i, 0, 0)),
        ],
        out_shape=[qkv_shape, qkv_shape, qkv_shape],
    )(xr, wqkvT, bqkv)

    grid_spec = pltpu.PrefetchScalarGridSpec(
        num_scalar_prefetch=1,
        grid=(TB,),
        in_specs=[
            pl.BlockSpec((1, L, C), lambda i, idx_ref: (i, 0, 0)),
            pl.BlockSpec((1, L, C), lambda i, idx_ref: (i, 0, 0)),
            pl.BlockSpec((1, L, C), lambda i, idx_ref: (i, 0, 0)),
            pl.BlockSpec((C, C), lambda i, idx_ref: (0, 0)),
            pl.BlockSpec((1, C), lambda i, idx_ref: (0, 0)),
        ],
        out_specs=pl.BlockSpec((1, L, C), lambda i, idx_ref: (i, 0, 0)),
        scratch_shapes=[
            pltpu.VMEM((TOPK * WIN, C), jnp.bfloat16),
            pltpu.VMEM((TOPK * WIN, C), jnp.bfloat16),
            pltpu.VMEM((WIN, C), jnp.float32),
        ],
    )
    out = pl.pallas_call(
        _attn_kernel,
        grid_spec=grid_spec,
        out_shape=jax.ShapeDtypeStruct((TB, L, C), jnp.float32),
    )(idx, q, k, v, wpT, bp)
    return out.reshape(T, B, L, C)
